# Initial kernel scaffold; baseline (speedup 1.0000x reference)
import jax
import jax.numpy as jnp
from jax.experimental import pallas as pl


def kernel(x, edge_index, W1, b1):
    raise NotImplementedError("write your pallas kernel here")


# speedup vs baseline: 23.1718x; 23.1718x over previous
import functools

import jax
import jax.numpy as jnp
from jax import lax
from jax.experimental import pallas as pl
from jax.experimental.pallas import tpu as pltpu
from jax.experimental.pallas import tpu_sc as plsc

N = 10000
E = 320000
F = 128
FH = F // 2
NC, NS = 2, 16
NW = NC * NS
CHUNK = 128
NCHUNK = 79
PER_W = CHUNK * NCHUNK
EPAD = NW * PER_W
NPAD = 10240
RPT = NPAD // NS

_mesh = plsc.VectorSubcoreMesh(core_axis_name="c", subcore_axis_name="s")
_sc_params = pltpu.CompilerParams(use_tc_tiling_on_sc=False)


@functools.partial(
    pl.kernel,
    out_type=jax.ShapeDtypeStruct((NC, NPAD, 16), jnp.float32),
    mesh=_mesh,
    compiler_params=_sc_params,
    scratch_types=[
        pltpu.VMEM((NCHUNK, CHUNK), jnp.int32),
        pltpu.VMEM((CHUNK, 16), jnp.float32),
        pltpu.VMEM((CHUNK, 16), jnp.float32),
        pltpu.VMEM_SHARED((NPAD, 16), jnp.float32),
    ],
)
def _sc_degree(dst_hbm, out_hbm, idx_v, ones_v, zero_v, deg_sh):
    cid = lax.axis_index("c")
    sid = lax.axis_index("s")
    wid = cid * NS + sid

    @pl.loop(0, CHUNK)
    def _(r):
        ones_v[r, :] = jnp.ones((16,), jnp.float32)
        zero_v[r, :] = jnp.zeros((16,), jnp.float32)

    base = sid * RPT
    @pl.loop(0, RPT // CHUNK)
    def _(z):
        pltpu.sync_copy(zero_v, deg_sh.at[pl.ds(base + z * CHUNK, CHUNK)])

    plsc.subcore_barrier()
    pltpu.sync_copy(dst_hbm.at[wid], idx_v)

    @pl.loop(0, NCHUNK)
    def _(j):
        pltpu.sync_copy(ones_v, deg_sh.at[idx_v.at[j]], add=True)

    plsc.subcore_barrier()
    pltpu.sync_copy(deg_sh.at[pl.ds(base, RPT)],
                    out_hbm.at[cid, pl.ds(base, RPT)])


@functools.partial(
    pl.kernel,
    out_type=jax.ShapeDtypeStruct((NC, 2, NPAD, FH), jnp.float32),
    mesh=_mesh,
    compiler_params=_sc_params,
    scratch_types=[
        pltpu.VMEM((NCHUNK, CHUNK), jnp.int32),
        pltpu.VMEM((NCHUNK, CHUNK), jnp.int32),
        pltpu.VMEM((CHUNK, FH), jnp.float32),
        pltpu.VMEM((CHUNK, FH), jnp.float32),
        pltpu.VMEM_SHARED((NPAD, FH), jnp.float32),
        pltpu.SemaphoreType.DMA,
    ],
)
def _sc_aggregate(g0_hbm, g1_hbm, src_hbm, dst_hbm, out_hbm,
                  si_v, di_v, rows_v, zero_v, acc_sh, sem):
    cid = lax.axis_index("c")
    sid = lax.axis_index("s")
    wid = cid * NS + sid
    base = sid * RPT

    @pl.loop(0, CHUNK)
    def _(r):
        @pl.loop(0, FH, step=16)
        def _(cc):
            zero_v[r, pl.ds(cc, 16)] = jnp.zeros((16,), jnp.float32)

    pltpu.sync_copy(src_hbm.at[wid], si_v)
    pltpu.sync_copy(dst_hbm.at[wid], di_v)

    for half, g_hbm in ((0, g0_hbm), (1, g1_hbm)):
        @pl.loop(0, RPT // CHUNK)
        def _(z):
            pltpu.sync_copy(zero_v, acc_sh.at[pl.ds(base + z * CHUNK, CHUNK)])

        plsc.subcore_barrier()

        @pl.loop(0, NCHUNK)
        def _(j):
            pltpu.async_copy(g_hbm.at[si_v.at[j]], rows_v, sem).wait()
            pltpu.sync_copy(rows_v, acc_sh.at[di_v.at[j]], add=True)

        plsc.subcore_barrier()
        pltpu.sync_copy(acc_sh.at[pl.ds(base, RPT)],
                        out_hbm.at[cid, half, pl.ds(base, RPT)])


def _tc_linear(x, W1, hist):
    def body(x_ref, w_ref, h_ref, g0_ref, g1_ref):
        deg = 1.0 + h_ref[0, :, 0] + h_ref[1, :, 0]
        dinv = lax.rsqrt(deg)
        h = jnp.dot(x_ref[...], w_ref[...],
                    preferred_element_type=jnp.float32,
                    precision=lax.Precision.HIGHEST)
        g = h * dinv[:, None]
        g0_ref[...] = g[:, :FH]
        g1_ref[...] = g[:, FH:]

    blk = 1000
    return pl.pallas_call(
        body,
        grid=(N // blk,),
        in_specs=[
            pl.BlockSpec((blk, F), lambda i: (i, 0)),
            pl.BlockSpec((F, F), lambda i: (0, 0)),
            pl.BlockSpec((NC, blk, 16), lambda i: (0, i, 0)),
        ],
        out_specs=[
            pl.BlockSpec((blk, FH), lambda i: (i, 0)),
            pl.BlockSpec((blk, FH), lambda i: (i, 0)),
        ],
        out_shape=[
            jax.ShapeDtypeStruct((N, FH), jnp.float32),
            jax.ShapeDtypeStruct((N, FH), jnp.float32),
        ],
    )(x, W1, hist)


def _tc_finish(p, g0, g1, hist, b2):
    def body(p_ref, g0_ref, g1_ref, h_ref, b_ref, o_ref):
        deg = 1.0 + h_ref[0, :, 0] + h_ref[1, :, 0]
        dinv = lax.rsqrt(deg)
        s0 = p_ref[0, 0] + p_ref[1, 0] + g0_ref[...]
        s1 = p_ref[0, 1] + p_ref[1, 1] + g1_ref[...]
        s = jnp.concatenate([s0, s1], axis=1)
        o_ref[...] = jnp.maximum(s * dinv[:, None] + b_ref[...], 0.0)

    blk = 1000
    return pl.pallas_call(
        body,
        grid=(N // blk,),
        in_specs=[
            pl.BlockSpec((NC, 2, blk, FH), lambda i: (0, 0, i, 0)),
            pl.BlockSpec((blk, FH), lambda i: (i, 0)),
            pl.BlockSpec((blk, FH), lambda i: (i, 0)),
            pl.BlockSpec((NC, blk, 16), lambda i: (0, i, 0)),
            pl.BlockSpec((1, F), lambda i: (0, 0)),
        ],
        out_specs=pl.BlockSpec((blk, F), lambda i: (i, 0)),
        out_shape=jax.ShapeDtypeStruct((N, F), jnp.float32),
    )(p, g0, g1, hist, b2)


def kernel(x, edge_index, W1, b1):
    src = edge_index[0].astype(jnp.int32)
    dst = edge_index[1].astype(jnp.int32)
    npad = EPAD - E
    ar = jnp.arange(npad, dtype=jnp.int32)
    src3 = jnp.concatenate([src, ar % 128]).reshape(NW, NCHUNK, CHUNK)
    dst3 = jnp.concatenate([dst, N + (ar % (NPAD - N))]).reshape(NW, NCHUNK, CHUNK)

    hist = _sc_degree(dst3)
    g0, g1 = _tc_linear(x, W1, hist)
    p = _sc_aggregate(g0, g1, src3, dst3)
    return _tc_finish(p, g0, g1, hist, b1.reshape(1, F))


# speedup vs baseline: 35.3104x; 1.5239x over previous
import functools

import jax
import jax.numpy as jnp
from jax import lax
from jax.experimental import pallas as pl
from jax.experimental.pallas import tpu as pltpu
from jax.experimental.pallas import tpu_sc as plsc

N = 10000
E = 320000
F = 128
FH = F // 2
NC, NS = 2, 16
NW = NC * NS
CHUNK = 128
NCHUNK = 80
NBUF = 4
PER_W = CHUNK * NCHUNK
EPAD = NW * PER_W
NPAD = 10240
RPT = NPAD // NS

_mesh = plsc.VectorSubcoreMesh(core_axis_name="c", subcore_axis_name="s")
_sc_params = pltpu.CompilerParams(use_tc_tiling_on_sc=False)


@functools.partial(
    pl.kernel,
    out_type=jax.ShapeDtypeStruct((NC, NPAD, 16), jnp.float32),
    mesh=_mesh,
    compiler_params=_sc_params,
    scratch_types=[
        pltpu.VMEM((NCHUNK, CHUNK), jnp.int32),
        pltpu.VMEM((CHUNK, 16), jnp.float32),
        pltpu.VMEM((CHUNK, 16), jnp.float32),
        pltpu.VMEM_SHARED((NPAD, 16), jnp.float32),
    ]
    + [pltpu.SemaphoreType.DMA] * NBUF,
)
def _sc_degree(dst_hbm, out_hbm, idx_v, ones_v, zero_v, deg_sh, *sems):
    cid = lax.axis_index("c")
    sid = lax.axis_index("s")
    wid = cid * NS + sid

    @pl.loop(0, CHUNK)
    def _(r):
        ones_v[r, :] = jnp.ones((16,), jnp.float32)
        zero_v[r, :] = jnp.zeros((16,), jnp.float32)

    base = sid * RPT
    @pl.loop(0, RPT // CHUNK)
    def _(z):
        pltpu.sync_copy(zero_v, deg_sh.at[pl.ds(base + z * CHUNK, CHUNK)])

    plsc.subcore_barrier()
    pltpu.sync_copy(dst_hbm.at[wid], idx_v)

    @pl.loop(0, NCHUNK, step=NBUF)
    def _(j):
        for b in range(NBUF):
            @pl.when(j > 0)
            def _():
                pltpu.make_async_copy(ones_v, deg_sh.at[idx_v.at[0]],
                                      sems[b]).wait()
            pltpu.async_copy(ones_v, deg_sh.at[idx_v.at[j + b]],
                             sems[b], add=True)

    for b in range(NBUF):
        pltpu.make_async_copy(ones_v, deg_sh.at[idx_v.at[0]], sems[b]).wait()

    plsc.subcore_barrier()
    pltpu.sync_copy(deg_sh.at[pl.ds(base, RPT)],
                    out_hbm.at[cid, pl.ds(base, RPT)])


@functools.partial(
    pl.kernel,
    out_type=jax.ShapeDtypeStruct((NC, 2, NPAD, FH), jnp.float32),
    mesh=_mesh,
    compiler_params=_sc_params,
    scratch_types=[
        pltpu.VMEM((NCHUNK, CHUNK), jnp.int32),
        pltpu.VMEM((NCHUNK, CHUNK), jnp.int32),
        pltpu.VMEM((CHUNK, FH), jnp.float32),
        pltpu.VMEM_SHARED((NPAD, FH), jnp.float32),
    ]
    + [pltpu.VMEM((CHUNK, FH), jnp.float32)] * NBUF
    + [pltpu.SemaphoreType.DMA] * (2 * NBUF),
)
def _sc_aggregate(g0_hbm, g1_hbm, src_hbm, dst_hbm, out_hbm,
                  si_v, di_v, zero_v, acc_sh, *bufs_and_sems):
    rows = bufs_and_sems[:NBUF]
    gsem = bufs_and_sems[NBUF:2 * NBUF]
    ssem = bufs_and_sems[2 * NBUF:]
    cid = lax.axis_index("c")
    sid = lax.axis_index("s")
    wid = cid * NS + sid
    base = sid * RPT

    @pl.loop(0, CHUNK)
    def _(r):
        @pl.loop(0, FH, step=16)
        def _(cc):
            zero_v[r, pl.ds(cc, 16)] = jnp.zeros((16,), jnp.float32)

    pltpu.sync_copy(src_hbm.at[wid], si_v)
    pltpu.sync_copy(dst_hbm.at[wid], di_v)

    for half, g_hbm in ((0, g0_hbm), (1, g1_hbm)):
        @pl.loop(0, RPT // CHUNK)
        def _(z):
            pltpu.sync_copy(zero_v, acc_sh.at[pl.ds(base + z * CHUNK, CHUNK)])

        plsc.subcore_barrier()

        @pl.loop(0, NCHUNK, step=NBUF)
        def _(j):
            for b in range(NBUF):
                @pl.when(j > 0)
                def _():
                    pltpu.make_async_copy(rows[b], acc_sh.at[di_v.at[0]],
                                          ssem[b]).wait()
                pltpu.async_copy(g_hbm.at[si_v.at[j + b]], rows[b], gsem[b])
            for b in range(NBUF):
                pltpu.make_async_copy(g_hbm.at[si_v.at[0]], rows[b],
                                      gsem[b]).wait()
                pltpu.async_copy(rows[b], acc_sh.at[di_v.at[j + b]],
                                 ssem[b], add=True)

        for b in range(NBUF):
            pltpu.make_async_copy(rows[b], acc_sh.at[di_v.at[0]],
                                  ssem[b]).wait()

        plsc.subcore_barrier()
        pltpu.sync_copy(acc_sh.at[pl.ds(base, RPT)],
                        out_hbm.at[cid, half, pl.ds(base, RPT)])
        plsc.subcore_barrier()


def _tc_matmul(x, W1):
    def body(x_ref, w_ref, h_ref):
        h_ref[...] = jnp.dot(x_ref[...], w_ref[...],
                             preferred_element_type=jnp.float32,
                             precision=lax.Precision.HIGHEST)

    blk = 1000
    return pl.pallas_call(
        body,
        grid=(N // blk,),
        in_specs=[
            pl.BlockSpec((blk, F), lambda i: (i, 0)),
            pl.BlockSpec((F, F), lambda i: (0, 0)),
        ],
        out_specs=pl.BlockSpec((blk, F), lambda i: (i, 0)),
        out_shape=jax.ShapeDtypeStruct((N, F), jnp.float32),
    )(x, W1)


def _tc_scale(h, hist):
    def body(h_ref, hi_ref, g0_ref, g1_ref):
        deg = 1.0 + hi_ref[0, :, 0] + hi_ref[1, :, 0]
        dinv = lax.rsqrt(deg)
        g = h_ref[...] * dinv[:, None]
        g0_ref[...] = g[:, :FH]
        g1_ref[...] = g[:, FH:]

    blk = 1000
    return pl.pallas_call(
        body,
        grid=(N // blk,),
        in_specs=[
            pl.BlockSpec((blk, F), lambda i: (i, 0)),
            pl.BlockSpec((NC, blk, 16), lambda i: (0, i, 0)),
        ],
        out_specs=[
            pl.BlockSpec((blk, FH), lambda i: (i, 0)),
            pl.BlockSpec((blk, FH), lambda i: (i, 0)),
        ],
        out_shape=[
            jax.ShapeDtypeStruct((N, FH), jnp.float32),
            jax.ShapeDtypeStruct((N, FH), jnp.float32),
        ],
    )(h, hist)


def _tc_finish(p, g0, g1, hist, b2):
    def body(p_ref, g0_ref, g1_ref, h_ref, b_ref, o_ref):
        deg = 1.0 + h_ref[0, :, 0] + h_ref[1, :, 0]
        dinv = lax.rsqrt(deg)
        s0 = p_ref[0, 0] + p_ref[1, 0] + g0_ref[...]
        s1 = p_ref[0, 1] + p_ref[1, 1] + g1_ref[...]
        s = jnp.concatenate([s0, s1], axis=1)
        o_ref[...] = jnp.maximum(s * dinv[:, None] + b_ref[...], 0.0)

    blk = 1000
    return pl.pallas_call(
        body,
        grid=(N // blk,),
        in_specs=[
            pl.BlockSpec((NC, 2, blk, FH), lambda i: (0, 0, i, 0)),
            pl.BlockSpec((blk, FH), lambda i: (i, 0)),
            pl.BlockSpec((blk, FH), lambda i: (i, 0)),
            pl.BlockSpec((NC, blk, 16), lambda i: (0, i, 0)),
            pl.BlockSpec((1, F), lambda i: (0, 0)),
        ],
        out_specs=pl.BlockSpec((blk, F), lambda i: (i, 0)),
        out_shape=jax.ShapeDtypeStruct((N, F), jnp.float32),
    )(p, g0, g1, hist, b2)


def kernel(x, edge_index, W1, b1):
    src = edge_index[0].astype(jnp.int32)
    dst = edge_index[1].astype(jnp.int32)
    npad = EPAD - E
    ar = jnp.arange(npad, dtype=jnp.int32)
    src3 = jnp.concatenate([src, ar % 128]).reshape(NW, NCHUNK, CHUNK)
    dst3 = jnp.concatenate([dst, N + (ar % (NPAD - N))]).reshape(NW, NCHUNK, CHUNK)

    hist = _sc_degree(dst3)
    h = _tc_matmul(x, W1)
    g0, g1 = _tc_scale(h, hist)
    p = _sc_aggregate(g0, g1, src3, dst3)
    return _tc_finish(p, g0, g1, hist, b1.reshape(1, F))


# speedup vs baseline: 36.2082x; 1.0254x over previous
import functools

import jax
import jax.numpy as jnp
from jax import lax
from jax.experimental import pallas as pl
from jax.experimental.pallas import tpu as pltpu
from jax.experimental.pallas import tpu_sc as plsc

N = 10000
E = 320000
F = 128
FH = F // 2
NC, NS = 2, 16
CHUNK = 128
NCHUNK = 160
NCHDEG = 80
NBUF = 4
ZB = 128
PER_T = CHUNK * NCHUNK
EPAD = NS * PER_T
NPAD = 10240
RPT = NPAD // NS

_mesh = plsc.VectorSubcoreMesh(core_axis_name="c", subcore_axis_name="s")
_sc_params = pltpu.CompilerParams(use_tc_tiling_on_sc=False)


@functools.partial(
    pl.kernel,
    out_type=jax.ShapeDtypeStruct((NC, NPAD, 16), jnp.float32),
    mesh=_mesh,
    compiler_params=_sc_params,
    scratch_types=[
        pltpu.VMEM((NCHDEG, CHUNK), jnp.int32),
        pltpu.VMEM((CHUNK, 16), jnp.float32),
        pltpu.VMEM((ZB, 16), jnp.float32),
        pltpu.VMEM_SHARED((NPAD, 16), jnp.float32),
    ]
    + [pltpu.SemaphoreType.DMA] * NBUF,
)
def _sc_degree(dst_hbm, out_hbm, idx_v, ones_v, zero_v, deg_sh, *sems):
    cid = lax.axis_index("c")
    sid = lax.axis_index("s")

    @pl.loop(0, CHUNK)
    def _(r):
        ones_v[r, :] = jnp.ones((16,), jnp.float32)

    @pl.loop(0, ZB)
    def _(r):
        zero_v[r, :] = jnp.zeros((16,), jnp.float32)

    base = sid * RPT
    @pl.loop(0, RPT // ZB)
    def _(z):
        pltpu.sync_copy(zero_v, deg_sh.at[pl.ds(base + z * ZB, ZB)])

    plsc.subcore_barrier()
    pltpu.sync_copy(dst_hbm.at[sid, pl.ds(cid * NCHDEG, NCHDEG)], idx_v)

    @pl.loop(0, NCHDEG, step=NBUF)
    def _(j):
        for b in range(NBUF):
            @pl.when(j > 0)
            def _():
                pltpu.make_async_copy(ones_v, deg_sh.at[idx_v.at[0]],
                                      sems[b]).wait()
            pltpu.async_copy(ones_v, deg_sh.at[idx_v.at[j + b]],
                             sems[b], add=True)

    for b in range(NBUF):
        pltpu.make_async_copy(ones_v, deg_sh.at[idx_v.at[0]], sems[b]).wait()

    plsc.subcore_barrier()
    pltpu.sync_copy(deg_sh.at[pl.ds(base, RPT)],
                    out_hbm.at[cid, pl.ds(base, RPT)])


@functools.partial(
    pl.kernel,
    out_type=jax.ShapeDtypeStruct((NC, NPAD, FH), jnp.float32),
    mesh=_mesh,
    compiler_params=_sc_params,
    scratch_types=[
        pltpu.VMEM((NCHUNK, CHUNK), jnp.int32),
        pltpu.VMEM((NCHUNK, CHUNK), jnp.int32),
        pltpu.VMEM((ZB, FH), jnp.float32),
        pltpu.VMEM_SHARED((NPAD, FH), jnp.float32),
    ]
    + [pltpu.VMEM((CHUNK, FH), jnp.float32)] * NBUF
    + [pltpu.SemaphoreType.DMA] * (2 * NBUF),
)
def _sc_aggregate(g_hbm, src_hbm, dst_hbm, out_hbm,
                  si_v, di_v, zero_v, acc_sh, *bufs_and_sems):
    rows = bufs_and_sems[:NBUF]
    gsem = bufs_and_sems[NBUF:2 * NBUF]
    ssem = bufs_and_sems[2 * NBUF:]
    cid = lax.axis_index("c")
    sid = lax.axis_index("s")
    base = sid * RPT

    @pl.loop(0, ZB)
    def _(r):
        @pl.loop(0, FH, step=16)
        def _(cc):
            zero_v[r, pl.ds(cc, 16)] = jnp.zeros((16,), jnp.float32)

    pltpu.sync_copy(src_hbm.at[cid, sid], si_v)
    pltpu.sync_copy(dst_hbm.at[sid], di_v)

    @pl.loop(0, RPT // ZB)
    def _(z):
        pltpu.sync_copy(zero_v, acc_sh.at[pl.ds(base + z * ZB, ZB)])

    plsc.subcore_barrier()

    @pl.loop(0, NCHUNK, step=NBUF)
    def _(j):
        for b in range(NBUF):
            @pl.when(j > 0)
            def _():
                pltpu.make_async_copy(rows[b], acc_sh.at[di_v.at[0]],
                                      ssem[b]).wait()
            pltpu.async_copy(g_hbm.at[si_v.at[j + b]], rows[b], gsem[b])
        for b in range(NBUF):
            pltpu.make_async_copy(g_hbm.at[si_v.at[0]], rows[b],
                                  gsem[b]).wait()
            pltpu.async_copy(rows[b], acc_sh.at[di_v.at[j + b]],
                             ssem[b], add=True)

    for b in range(NBUF):
        pltpu.make_async_copy(rows[b], acc_sh.at[di_v.at[0]],
                              ssem[b]).wait()

    plsc.subcore_barrier()
    pltpu.sync_copy(acc_sh.at[pl.ds(base, RPT)],
                    out_hbm.at[cid, pl.ds(base, RPT)])


def _tc_matmul(x, W1):
    def body(x_ref, w_ref, h_ref):
        h_ref[...] = jnp.dot(x_ref[...], w_ref[...],
                             preferred_element_type=jnp.float32,
                             precision=lax.Precision.HIGHEST)

    blk = 1000
    return pl.pallas_call(
        body,
        grid=(N // blk,),
        in_specs=[
            pl.BlockSpec((blk, F), lambda i: (i, 0)),
            pl.BlockSpec((F, F), lambda i: (0, 0)),
        ],
        out_specs=pl.BlockSpec((blk, F), lambda i: (i, 0)),
        out_shape=jax.ShapeDtypeStruct((N, F), jnp.float32),
    )(x, W1)


def _tc_scale(h, hist):
    def body(h_ref, hi_ref, g_ref):
        deg = 1.0 + hi_ref[0, :, 0] + hi_ref[1, :, 0]
        dinv = lax.rsqrt(deg)
        g = h_ref[...] * dinv[:, None]
        g_ref[0] = g[:, :FH]
        g_ref[1] = g[:, FH:]

    blk = 1000
    return pl.pallas_call(
        body,
        grid=(N // blk,),
        in_specs=[
            pl.BlockSpec((blk, F), lambda i: (i, 0)),
            pl.BlockSpec((NC, blk, 16), lambda i: (0, i, 0)),
        ],
        out_specs=pl.BlockSpec((2, blk, FH), lambda i: (0, i, 0)),
        out_shape=jax.ShapeDtypeStruct((2, N, FH), jnp.float32),
    )(h, hist)


def _tc_finish(p, g, hist, b2):
    def body(p_ref, g_ref, h_ref, b_ref, o_ref):
        deg = 1.0 + h_ref[0, :, 0] + h_ref[1, :, 0]
        dinv = lax.rsqrt(deg)
        s0 = p_ref[0] + g_ref[0]
        s1 = p_ref[1] + g_ref[1]
        s = jnp.concatenate([s0, s1], axis=1)
        o_ref[...] = jnp.maximum(s * dinv[:, None] + b_ref[...], 0.0)

    blk = 1000
    return pl.pallas_call(
        body,
        grid=(N // blk,),
        in_specs=[
            pl.BlockSpec((NC, blk, FH), lambda i: (0, i, 0)),
            pl.BlockSpec((2, blk, FH), lambda i: (0, i, 0)),
            pl.BlockSpec((NC, blk, 16), lambda i: (0, i, 0)),
            pl.BlockSpec((1, F), lambda i: (0, 0)),
        ],
        out_specs=pl.BlockSpec((blk, F), lambda i: (i, 0)),
        out_shape=jax.ShapeDtypeStruct((N, F), jnp.float32),
    )(p, g, hist, b2)


def kernel(x, edge_index, W1, b1):
    src = edge_index[0].astype(jnp.int32)
    dst = edge_index[1].astype(jnp.int32)
    npad = EPAD - E
    ar = jnp.arange(npad, dtype=jnp.int32)
    srcp = jnp.concatenate([src, ar % 128])
    srcb = jnp.stack([srcp, srcp + N]).reshape(NC, NS, NCHUNK, CHUNK)
    dst3 = jnp.concatenate([dst, N + (ar % (NPAD - N))]).reshape(NS, NCHUNK, CHUNK)

    hist = _sc_degree(dst3)
    h = _tc_matmul(x, W1)
    g = _tc_scale(h, hist)
    p = _sc_aggregate(g.reshape(2 * N, FH), srcb, dst3)
    return _tc_finish(p, g, hist, b1.reshape(1, F))


# speedup vs baseline: 39.4469x; 1.0894x over previous
import functools

import jax
import jax.numpy as jnp
from jax import lax
from jax.experimental import pallas as pl
from jax.experimental.pallas import tpu as pltpu
from jax.experimental.pallas import tpu_sc as plsc

N = 10000
E = 320000
F = 128
FH = F // 2
NC, NS = 2, 16
CHUNK = 128
NCHUNK = 160
NCHDEG = 80
NBUF = 4
ZB = 128
PER_T = CHUNK * NCHUNK
EPAD = NS * PER_T
NPAD = 10240
RPT = NPAD // NS

_mesh = plsc.VectorSubcoreMesh(core_axis_name="c", subcore_axis_name="s")
_sc_params = pltpu.CompilerParams(use_tc_tiling_on_sc=False)


@functools.partial(
    pl.kernel,
    out_type=jax.ShapeDtypeStruct((NC, NPAD, 16), jnp.float32),
    mesh=_mesh,
    compiler_params=_sc_params,
    scratch_types=[
        pltpu.VMEM((NCHDEG, CHUNK), jnp.int32),
        pltpu.VMEM((CHUNK, 16), jnp.float32),
        pltpu.VMEM((ZB, 16), jnp.float32),
        pltpu.VMEM_SHARED((NPAD, 16), jnp.float32),
    ]
    + [pltpu.SemaphoreType.DMA] * NBUF,
)
def _sc_degree(dst_hbm, out_hbm, idx_v, ones_v, zero_v, deg_sh, *sems):
    cid = lax.axis_index("c")
    sid = lax.axis_index("s")

    @pl.loop(0, CHUNK)
    def _(r):
        ones_v[r, :] = jnp.ones((16,), jnp.float32)

    @pl.loop(0, ZB)
    def _(r):
        zero_v[r, :] = jnp.zeros((16,), jnp.float32)

    base = sid * RPT
    @pl.loop(0, RPT // ZB)
    def _(z):
        pltpu.sync_copy(zero_v, deg_sh.at[pl.ds(base + z * ZB, ZB)])

    plsc.subcore_barrier()
    pltpu.sync_copy(dst_hbm.at[sid, pl.ds(cid * NCHDEG, NCHDEG)], idx_v)

    @pl.loop(0, NCHDEG, step=NBUF)
    def _(j):
        for b in range(NBUF):
            @pl.when(j > 0)
            def _():
                pltpu.make_async_copy(ones_v, deg_sh.at[idx_v.at[0]],
                                      sems[b]).wait()
            pltpu.async_copy(ones_v, deg_sh.at[idx_v.at[j + b]],
                             sems[b], add=True)

    for b in range(NBUF):
        pltpu.make_async_copy(ones_v, deg_sh.at[idx_v.at[0]], sems[b]).wait()

    plsc.subcore_barrier()
    pltpu.sync_copy(deg_sh.at[pl.ds(base, RPT)],
                    out_hbm.at[cid, pl.ds(base, RPT)])


@functools.partial(
    pl.kernel,
    out_type=jax.ShapeDtypeStruct((NC, NPAD, FH), jnp.float32),
    mesh=_mesh,
    compiler_params=_sc_params,
    scratch_types=[
        pltpu.VMEM((NCHUNK, CHUNK), jnp.int32),
        pltpu.VMEM((NCHUNK, CHUNK), jnp.int32),
        pltpu.VMEM((ZB, FH), jnp.float32),
        pltpu.VMEM_SHARED((NPAD, FH), jnp.float32),
    ]
    + [pltpu.VMEM((CHUNK, FH), jnp.float32)] * NBUF
    + [pltpu.SemaphoreType.DMA] * (2 * NBUF),
)
def _sc_aggregate(g_hbm, src_hbm, dst_hbm, out_hbm,
                  si_v, di_v, zero_v, acc_sh, *bufs_and_sems):
    rows = bufs_and_sems[:NBUF]
    gsem = bufs_and_sems[NBUF:2 * NBUF]
    ssem = bufs_and_sems[2 * NBUF:]
    cid = lax.axis_index("c")
    sid = lax.axis_index("s")
    base = sid * RPT

    @pl.loop(0, ZB)
    def _(r):
        @pl.loop(0, FH, step=16)
        def _(cc):
            zero_v[r, pl.ds(cc, 16)] = jnp.zeros((16,), jnp.float32)

    pltpu.sync_copy(src_hbm.at[cid, sid], si_v)
    pltpu.sync_copy(dst_hbm.at[sid], di_v)

    @pl.loop(0, RPT // ZB)
    def _(z):
        pltpu.sync_copy(zero_v, acc_sh.at[pl.ds(base + z * ZB, ZB)])

    plsc.subcore_barrier()

    @pl.loop(0, NCHUNK, step=NBUF)
    def _(j):
        for b in range(NBUF):
            @pl.when(j > 0)
            def _():
                pltpu.make_async_copy(rows[b], acc_sh.at[di_v.at[0]],
                                      ssem[b]).wait()
            pltpu.async_copy(g_hbm.at[si_v.at[j + b]], rows[b], gsem[b])
        for b in range(NBUF):
            pltpu.make_async_copy(g_hbm.at[si_v.at[0]], rows[b],
                                  gsem[b]).wait()
            pltpu.async_copy(rows[b], acc_sh.at[di_v.at[j + b]],
                             ssem[b], add=True)

    for b in range(NBUF):
        pltpu.make_async_copy(rows[b], acc_sh.at[di_v.at[0]],
                              ssem[b]).wait()

    plsc.subcore_barrier()
    pltpu.sync_copy(acc_sh.at[pl.ds(base, RPT)],
                    out_hbm.at[cid, pl.ds(base, RPT)])


def _tc_matmul(x, W1):
    def body(x_ref, w_ref, h_ref):
        h_ref[...] = jnp.dot(x_ref[...], w_ref[...],
                             preferred_element_type=jnp.float32,
                             precision=lax.Precision.HIGHEST)

    blk = 1000
    return pl.pallas_call(
        body,
        grid=(N // blk,),
        in_specs=[
            pl.BlockSpec((blk, F), lambda i: (i, 0)),
            pl.BlockSpec((F, F), lambda i: (0, 0)),
        ],
        out_specs=pl.BlockSpec((blk, F), lambda i: (i, 0)),
        out_shape=jax.ShapeDtypeStruct((N, F), jnp.float32),
    )(x, W1)


def _tc_scale(h, hist):
    def body(h_ref, hi_ref, g_ref):
        deg = 1.0 + hi_ref[0, :, 0] + hi_ref[1, :, 0]
        dinv = lax.rsqrt(deg)
        g = h_ref[...] * dinv[:, None]
        g_ref[0] = g[:, :FH]
        g_ref[1] = g[:, FH:]

    blk = 1000
    return pl.pallas_call(
        body,
        grid=(N // blk,),
        in_specs=[
            pl.BlockSpec((blk, F), lambda i: (i, 0)),
            pl.BlockSpec((NC, blk, 16), lambda i: (0, i, 0)),
        ],
        out_specs=pl.BlockSpec((2, blk, FH), lambda i: (0, i, 0)),
        out_shape=jax.ShapeDtypeStruct((2, N, FH), jnp.float32),
    )(h, hist)


def _tc_finish(p, g, hist, b2):
    def body(p_ref, g_ref, h_ref, b_ref, o_ref):
        deg = 1.0 + h_ref[0, :, 0] + h_ref[1, :, 0]
        dinv = lax.rsqrt(deg)
        s0 = p_ref[0] + g_ref[0]
        s1 = p_ref[1] + g_ref[1]
        s = jnp.concatenate([s0, s1], axis=1)
        o_ref[...] = jnp.maximum(s * dinv[:, None] + b_ref[...], 0.0)

    blk = 1000
    return pl.pallas_call(
        body,
        grid=(N // blk,),
        in_specs=[
            pl.BlockSpec((NC, blk, FH), lambda i: (0, i, 0)),
            pl.BlockSpec((2, blk, FH), lambda i: (0, i, 0)),
            pl.BlockSpec((NC, blk, 16), lambda i: (0, i, 0)),
            pl.BlockSpec((1, F), lambda i: (0, 0)),
        ],
        out_specs=pl.BlockSpec((blk, F), lambda i: (i, 0)),
        out_shape=jax.ShapeDtypeStruct((N, F), jnp.float32),
    )(p, g, hist, b2)


def kernel(x, edge_index, W1, b1):
    npad = EPAD - E
    ei = edge_index.astype(jnp.int32).reshape(2, E // 128, 128)
    ar = jnp.arange(npad, dtype=jnp.int32).reshape(npad // 128, 128)
    pads = jnp.stack([ar % 128, N + (ar % (NPAD - N))])
    eip = jnp.concatenate([ei, pads], axis=1)
    srcb = jnp.stack([eip[0], eip[0] + N]).reshape(NC, NS, NCHUNK, CHUNK)
    dst3 = eip[1].reshape(NS, NCHUNK, CHUNK)

    hist = _sc_degree(dst3)
    h = _tc_matmul(x, W1)
    g = _tc_scale(h, hist)
    p = _sc_aggregate(g.reshape(2 * N, FH), srcb, dst3)
    return _tc_finish(p, g, hist, b1.reshape(1, F))


# speedup vs baseline: 39.9752x; 1.0134x over previous
import functools

import jax
import jax.numpy as jnp
from jax import lax
from jax.experimental import pallas as pl
from jax.experimental.pallas import tpu as pltpu
from jax.experimental.pallas import tpu_sc as plsc

N = 10000
E = 320000
F = 128
FH = F // 2
NC, NS = 2, 16
CHUNK = 128
NCHUNK = 160
NCHDEG = 80
NBUF = 5
NBUFD = 8
ZB = 128
PER_T = CHUNK * NCHUNK
EPAD = NS * PER_T
NPAD = 10240
RPT = NPAD // NS

_mesh = plsc.VectorSubcoreMesh(core_axis_name="c", subcore_axis_name="s")
_sc_params = pltpu.CompilerParams(use_tc_tiling_on_sc=False)


@functools.partial(
    pl.kernel,
    out_type=jax.ShapeDtypeStruct((NC, NPAD, 16), jnp.float32),
    mesh=_mesh,
    compiler_params=_sc_params,
    scratch_types=[
        pltpu.VMEM((NCHDEG, CHUNK), jnp.int32),
        pltpu.VMEM((CHUNK, 16), jnp.float32),
        pltpu.VMEM((ZB, 16), jnp.float32),
        pltpu.VMEM_SHARED((NPAD, 16), jnp.float32),
    ]
    + [pltpu.SemaphoreType.DMA] * NBUFD,
)
def _sc_degree(dst_hbm, out_hbm, idx_v, ones_v, zero_v, deg_sh, *sems):
    cid = lax.axis_index("c")
    sid = lax.axis_index("s")

    @pl.loop(0, CHUNK)
    def _(r):
        ones_v[r, :] = jnp.ones((16,), jnp.float32)

    @pl.loop(0, ZB)
    def _(r):
        zero_v[r, :] = jnp.zeros((16,), jnp.float32)

    base = sid * RPT
    @pl.loop(0, RPT // ZB)
    def _(z):
        pltpu.sync_copy(zero_v, deg_sh.at[pl.ds(base + z * ZB, ZB)])

    plsc.subcore_barrier()
    pltpu.sync_copy(dst_hbm.at[sid, pl.ds(cid * NCHDEG, NCHDEG)], idx_v)

    @pl.loop(0, NCHDEG, step=NBUFD)
    def _(j):
        for b in range(NBUFD):
            @pl.when(j > 0)
            def _():
                pltpu.make_async_copy(ones_v, deg_sh.at[idx_v.at[0]],
                                      sems[b]).wait()
            pltpu.async_copy(ones_v, deg_sh.at[idx_v.at[j + b]],
                             sems[b], add=True)

    for b in range(NBUFD):
        pltpu.make_async_copy(ones_v, deg_sh.at[idx_v.at[0]], sems[b]).wait()

    plsc.subcore_barrier()
    pltpu.sync_copy(deg_sh.at[pl.ds(base, RPT)],
                    out_hbm.at[cid, pl.ds(base, RPT)])


@functools.partial(
    pl.kernel,
    out_type=jax.ShapeDtypeStruct((NC, NPAD, FH), jnp.float32),
    mesh=_mesh,
    compiler_params=_sc_params,
    scratch_types=[
        pltpu.VMEM((NCHUNK, CHUNK), jnp.int32),
        pltpu.VMEM((NCHUNK, CHUNK), jnp.int32),
        pltpu.VMEM((ZB, FH), jnp.float32),
        pltpu.VMEM_SHARED((NPAD, FH), jnp.float32),
    ]
    + [pltpu.VMEM((CHUNK, FH), jnp.float32)] * NBUF
    + [pltpu.SemaphoreType.DMA] * (2 * NBUF),
)
def _sc_aggregate(g_hbm, src_hbm, dst_hbm, out_hbm,
                  si_v, di_v, zero_v, acc_sh, *bufs_and_sems):
    rows = bufs_and_sems[:NBUF]
    gsem = bufs_and_sems[NBUF:2 * NBUF]
    ssem = bufs_and_sems[2 * NBUF:]
    cid = lax.axis_index("c")
    sid = lax.axis_index("s")
    base = sid * RPT

    @pl.loop(0, ZB)
    def _(r):
        @pl.loop(0, FH, step=16)
        def _(cc):
            zero_v[r, pl.ds(cc, 16)] = jnp.zeros((16,), jnp.float32)

    pltpu.sync_copy(src_hbm.at[cid, sid], si_v)
    pltpu.sync_copy(dst_hbm.at[sid], di_v)

    @pl.loop(0, RPT // ZB)
    def _(z):
        pltpu.sync_copy(zero_v, acc_sh.at[pl.ds(base + z * ZB, ZB)])

    plsc.subcore_barrier()

    @pl.loop(0, NCHUNK, step=NBUF)
    def _(j):
        for b in range(NBUF):
            @pl.when(j > 0)
            def _():
                pltpu.make_async_copy(rows[b], acc_sh.at[di_v.at[0]],
                                      ssem[b]).wait()
            pltpu.async_copy(g_hbm.at[si_v.at[j + b]], rows[b], gsem[b])
        for b in range(NBUF):
            pltpu.make_async_copy(g_hbm.at[si_v.at[0]], rows[b],
                                  gsem[b]).wait()
            pltpu.async_copy(rows[b], acc_sh.at[di_v.at[j + b]],
                             ssem[b], add=True)

    for b in range(NBUF):
        pltpu.make_async_copy(rows[b], acc_sh.at[di_v.at[0]],
                              ssem[b]).wait()

    plsc.subcore_barrier()
    pltpu.sync_copy(acc_sh.at[pl.ds(base, RPT)],
                    out_hbm.at[cid, pl.ds(base, RPT)])


def _tc_matmul(x, W1):
    def body(x_ref, w_ref, h_ref):
        h_ref[...] = jnp.dot(x_ref[...], w_ref[...],
                             preferred_element_type=jnp.float32,
                             precision=lax.Precision.HIGHEST)

    blk = 1000
    return pl.pallas_call(
        body,
        grid=(N // blk,),
        in_specs=[
            pl.BlockSpec((blk, F), lambda i: (i, 0)),
            pl.BlockSpec((F, F), lambda i: (0, 0)),
        ],
        out_specs=pl.BlockSpec((blk, F), lambda i: (i, 0)),
        out_shape=jax.ShapeDtypeStruct((N, F), jnp.float32),
    )(x, W1)


def _tc_scale(h, hist):
    def body(h_ref, hi_ref, g_ref):
        deg = 1.0 + hi_ref[0, :, 0] + hi_ref[1, :, 0]
        dinv = lax.rsqrt(deg)
        g = h_ref[...] * dinv[:, None]
        g_ref[0] = g[:, :FH]
        g_ref[1] = g[:, FH:]

    blk = 1000
    return pl.pallas_call(
        body,
        grid=(N // blk,),
        in_specs=[
            pl.BlockSpec((blk, F), lambda i: (i, 0)),
            pl.BlockSpec((NC, blk, 16), lambda i: (0, i, 0)),
        ],
        out_specs=pl.BlockSpec((2, blk, FH), lambda i: (0, i, 0)),
        out_shape=jax.ShapeDtypeStruct((2, N, FH), jnp.float32),
    )(h, hist)


def _tc_finish(p, g, hist, b2):
    def body(p_ref, g_ref, h_ref, b_ref, o_ref):
        deg = 1.0 + h_ref[0, :, 0] + h_ref[1, :, 0]
        dinv = lax.rsqrt(deg)
        s0 = p_ref[0] + g_ref[0]
        s1 = p_ref[1] + g_ref[1]
        s = jnp.concatenate([s0, s1], axis=1)
        o_ref[...] = jnp.maximum(s * dinv[:, None] + b_ref[...], 0.0)

    blk = 1000
    return pl.pallas_call(
        body,
        grid=(N // blk,),
        in_specs=[
            pl.BlockSpec((NC, blk, FH), lambda i: (0, i, 0)),
            pl.BlockSpec((2, blk, FH), lambda i: (0, i, 0)),
            pl.BlockSpec((NC, blk, 16), lambda i: (0, i, 0)),
            pl.BlockSpec((1, F), lambda i: (0, 0)),
        ],
        out_specs=pl.BlockSpec((blk, F), lambda i: (i, 0)),
        out_shape=jax.ShapeDtypeStruct((N, F), jnp.float32),
    )(p, g, hist, b2)


def kernel(x, edge_index, W1, b1):
    npad = EPAD - E
    ei = edge_index.astype(jnp.int32).reshape(2, E // 128, 128)
    ar = jnp.arange(npad, dtype=jnp.int32).reshape(npad // 128, 128)
    pads = jnp.stack([ar % 128, N + (ar % (NPAD - N))])
    eip = jnp.concatenate([ei, pads], axis=1)
    srcb = jnp.stack([eip[0], eip[0] + N]).reshape(NC, NS, NCHUNK, CHUNK)
    dst3 = eip[1].reshape(NS, NCHUNK, CHUNK)

    hist = _sc_degree(dst3)
    h = _tc_matmul(x, W1)
    g = _tc_scale(h, hist)
    p = _sc_aggregate(g.reshape(2 * N, FH), srcb, dst3)
    return _tc_finish(p, g, hist, b1.reshape(1, F))


# speedup vs baseline: 41.1393x; 1.0291x over previous
import functools

import jax
import jax.numpy as jnp
from jax import lax
from jax.experimental import pallas as pl
from jax.experimental.pallas import tpu as pltpu
from jax.experimental.pallas import tpu_sc as plsc

N = 10000
E = 320000
F = 128
FH = F // 2
NC, NS = 2, 16
CHUNK = 128
NCHUNK = 160
NCHDEG = 80
NBUF = 5
NBUFD = 8
ZB = 128
PER_T = CHUNK * NCHUNK
EPAD = NS * PER_T
NPAD = 10240
RPT = NPAD // NS

_mesh = plsc.VectorSubcoreMesh(core_axis_name="c", subcore_axis_name="s")
_sc_params = pltpu.CompilerParams(use_tc_tiling_on_sc=False)


@functools.partial(
    pl.kernel,
    out_type=jax.ShapeDtypeStruct((NC, NPAD, 16), jnp.float32),
    mesh=_mesh,
    compiler_params=_sc_params,
    scratch_types=[
        pltpu.VMEM((NCHDEG, CHUNK), jnp.int32),
        pltpu.VMEM((CHUNK, 16), jnp.float32),
        pltpu.VMEM((ZB, 16), jnp.float32),
        pltpu.VMEM_SHARED((NPAD, 16), jnp.float32),
    ]
    + [pltpu.SemaphoreType.DMA] * NBUFD,
)
def _sc_degree(dst_hbm, out_hbm, idx_v, ones_v, zero_v, deg_sh, *sems):
    cid = lax.axis_index("c")
    sid = lax.axis_index("s")

    @pl.loop(0, CHUNK)
    def _(r):
        ones_v[r, :] = jnp.ones((16,), jnp.float32)

    @pl.loop(0, ZB)
    def _(r):
        zero_v[r, :] = jnp.zeros((16,), jnp.float32)

    base = sid * RPT
    @pl.loop(0, RPT // ZB)
    def _(z):
        pltpu.sync_copy(zero_v, deg_sh.at[pl.ds(base + z * ZB, ZB)])

    plsc.subcore_barrier()
    pltpu.sync_copy(dst_hbm.at[sid, pl.ds(cid * NCHDEG, NCHDEG)], idx_v)

    @pl.loop(0, NCHDEG, step=NBUFD)
    def _(j):
        for b in range(NBUFD):
            @pl.when(j > 0)
            def _():
                pltpu.make_async_copy(ones_v, deg_sh.at[idx_v.at[0]],
                                      sems[b]).wait()
            pltpu.async_copy(ones_v, deg_sh.at[idx_v.at[j + b]],
                             sems[b], add=True)

    for b in range(NBUFD):
        pltpu.make_async_copy(ones_v, deg_sh.at[idx_v.at[0]], sems[b]).wait()

    plsc.subcore_barrier()
    pltpu.sync_copy(deg_sh.at[pl.ds(base, RPT)],
                    out_hbm.at[cid, pl.ds(base, RPT)])


@functools.partial(
    pl.kernel,
    out_type=jax.ShapeDtypeStruct((NC, NPAD, FH), jnp.float32),
    mesh=_mesh,
    compiler_params=_sc_params,
    scratch_types=[
        pltpu.VMEM((NCHUNK, CHUNK), jnp.int32),
        pltpu.VMEM((NCHUNK, CHUNK), jnp.int32),
        pltpu.VMEM((ZB, FH), jnp.float32),
        pltpu.VMEM_SHARED((NPAD, FH), jnp.float32),
    ]
    + [pltpu.VMEM((CHUNK, FH), jnp.float32)] * NBUF
    + [pltpu.SemaphoreType.DMA] * (2 * NBUF),
)
def _sc_aggregate(g_hbm, src_hbm, dst_hbm, out_hbm,
                  si_v, di_v, zero_v, acc_sh, *bufs_and_sems):
    rows = bufs_and_sems[:NBUF]
    gsem = bufs_and_sems[NBUF:2 * NBUF]
    ssem = bufs_and_sems[2 * NBUF:]
    cid = lax.axis_index("c")
    sid = lax.axis_index("s")
    base = sid * RPT

    @pl.loop(0, ZB)
    def _(r):
        @pl.loop(0, FH, step=16)
        def _(cc):
            zero_v[r, pl.ds(cc, 16)] = jnp.zeros((16,), jnp.float32)

    pltpu.sync_copy(src_hbm.at[cid, sid], si_v)
    pltpu.sync_copy(dst_hbm.at[sid], di_v)

    @pl.loop(0, RPT // ZB)
    def _(z):
        pltpu.sync_copy(zero_v, acc_sh.at[pl.ds(base + z * ZB, ZB)])

    plsc.subcore_barrier()

    @pl.loop(0, NCHUNK, step=NBUF)
    def _(j):
        for b in range(NBUF):
            @pl.when(j > 0)
            def _():
                pltpu.make_async_copy(rows[b], acc_sh.at[di_v.at[0]],
                                      ssem[b]).wait()
            pltpu.async_copy(g_hbm.at[si_v.at[j + b]], rows[b], gsem[b])
        for b in range(NBUF):
            pltpu.make_async_copy(g_hbm.at[si_v.at[0]], rows[b],
                                  gsem[b]).wait()
            pltpu.async_copy(rows[b], acc_sh.at[di_v.at[j + b]],
                             ssem[b], add=True)

    for b in range(NBUF):
        pltpu.make_async_copy(rows[b], acc_sh.at[di_v.at[0]],
                              ssem[b]).wait()

    plsc.subcore_barrier()
    pltpu.sync_copy(acc_sh.at[pl.ds(base, RPT)],
                    out_hbm.at[cid, pl.ds(base, RPT)])


def _tc_matmul(x, W1):
    def body(x_ref, w_ref, h_ref):
        h_ref[...] = jnp.dot(x_ref[...], w_ref[...],
                             preferred_element_type=jnp.float32,
                             precision=lax.Precision.HIGHEST)

    blk = 2000
    return pl.pallas_call(
        body,
        grid=(N // blk,),
        in_specs=[
            pl.BlockSpec((blk, F), lambda i: (i, 0)),
            pl.BlockSpec((F, F), lambda i: (0, 0)),
        ],
        out_specs=pl.BlockSpec((blk, F), lambda i: (i, 0)),
        out_shape=jax.ShapeDtypeStruct((N, F), jnp.float32),
    )(x, W1)


def _tc_scale(h, hist):
    def body(h_ref, hi_ref, g_ref):
        deg = 1.0 + hi_ref[0, :, 0] + hi_ref[1, :, 0]
        dinv = lax.rsqrt(deg)
        g = h_ref[...] * dinv[:, None]
        g_ref[0] = g[:, :FH]
        g_ref[1] = g[:, FH:]

    blk = 2000
    return pl.pallas_call(
        body,
        grid=(N // blk,),
        in_specs=[
            pl.BlockSpec((blk, F), lambda i: (i, 0)),
            pl.BlockSpec((NC, blk, 16), lambda i: (0, i, 0)),
        ],
        out_specs=pl.BlockSpec((2, blk, FH), lambda i: (0, i, 0)),
        out_shape=jax.ShapeDtypeStruct((2, N, FH), jnp.float32),
    )(h, hist)


def _tc_finish(p, g, hist, b2):
    def body(p_ref, g_ref, h_ref, b_ref, o_ref):
        deg = 1.0 + h_ref[0, :, 0] + h_ref[1, :, 0]
        dinv = lax.rsqrt(deg)
        s0 = p_ref[0] + g_ref[0]
        s1 = p_ref[1] + g_ref[1]
        s = jnp.concatenate([s0, s1], axis=1)
        o_ref[...] = jnp.maximum(s * dinv[:, None] + b_ref[...], 0.0)

    blk = 2000
    return pl.pallas_call(
        body,
        grid=(N // blk,),
        in_specs=[
            pl.BlockSpec((NC, blk, FH), lambda i: (0, i, 0)),
            pl.BlockSpec((2, blk, FH), lambda i: (0, i, 0)),
            pl.BlockSpec((NC, blk, 16), lambda i: (0, i, 0)),
            pl.BlockSpec((1, F), lambda i: (0, 0)),
        ],
        out_specs=pl.BlockSpec((blk, F), lambda i: (i, 0)),
        out_shape=jax.ShapeDtypeStruct((N, F), jnp.float32),
    )(p, g, hist, b2)


def kernel(x, edge_index, W1, b1):
    npad = EPAD - E
    ei = edge_index.astype(jnp.int32).reshape(2, E // 128, 128)
    ar = jnp.arange(npad, dtype=jnp.int32).reshape(npad // 128, 128)
    pads = jnp.stack([ar % 128, N + (ar % (NPAD - N))])
    eip = jnp.concatenate([ei, pads], axis=1)
    srcb = jnp.stack([eip[0], eip[0] + N]).reshape(NC, NS, NCHUNK, CHUNK)
    dst3 = eip[1].reshape(NS, NCHUNK, CHUNK)

    hist = _sc_degree(dst3)
    h = _tc_matmul(x, W1)
    g = _tc_scale(h, hist)
    p = _sc_aggregate(g.reshape(2 * N, FH), srcb, dst3)
    return _tc_finish(p, g, hist, b1.reshape(1, F))


# speedup vs baseline: 43.1376x; 1.0486x over previous
import functools

import jax
import jax.numpy as jnp
from jax import lax
from jax.experimental import pallas as pl
from jax.experimental.pallas import tpu as pltpu
from jax.experimental.pallas import tpu_sc as plsc

N = 10000
E = 320000
F = 128
FH = F // 2
NC, NS = 2, 16
CHUNK = 128
NCHUNK = 160
NCHDEG = 80
NBUF = 5
NBUFD = 8
ZB = 128
PER_T = CHUNK * NCHUNK
EPAD = NS * PER_T
NPAD = 10240
RPT = NPAD // NS

_mesh = plsc.VectorSubcoreMesh(core_axis_name="c", subcore_axis_name="s")
_sc_params = pltpu.CompilerParams(use_tc_tiling_on_sc=False)


@functools.partial(
    pl.kernel,
    out_type=jax.ShapeDtypeStruct((NC, NPAD, 16), jnp.float32),
    mesh=_mesh,
    compiler_params=_sc_params,
    scratch_types=[
        pltpu.VMEM((NCHDEG, CHUNK), jnp.int32),
        pltpu.VMEM((CHUNK, 16), jnp.float32),
        pltpu.VMEM((ZB, 16), jnp.float32),
        pltpu.VMEM_SHARED((NPAD, 16), jnp.float32),
    ]
    + [pltpu.SemaphoreType.DMA] * NBUFD,
)
def _sc_degree(dst_hbm, out_hbm, idx_v, ones_v, zero_v, deg_sh, *sems):
    cid = lax.axis_index("c")
    sid = lax.axis_index("s")

    @pl.loop(0, CHUNK)
    def _(r):
        ones_v[r, :] = jnp.ones((16,), jnp.float32)

    @pl.loop(0, ZB)
    def _(r):
        zero_v[r, :] = jnp.zeros((16,), jnp.float32)

    base = sid * RPT
    @pl.loop(0, RPT // ZB)
    def _(z):
        pltpu.sync_copy(zero_v, deg_sh.at[pl.ds(base + z * ZB, ZB)])

    plsc.subcore_barrier()
    pltpu.sync_copy(dst_hbm.at[sid, pl.ds(cid * NCHDEG, NCHDEG)], idx_v)

    @pl.loop(0, NCHDEG, step=NBUFD)
    def _(j):
        for b in range(NBUFD):
            @pl.when(j > 0)
            def _():
                pltpu.make_async_copy(ones_v, deg_sh.at[idx_v.at[0]],
                                      sems[b]).wait()
            pltpu.async_copy(ones_v, deg_sh.at[idx_v.at[j + b]],
                             sems[b], add=True)

    for b in range(NBUFD):
        pltpu.make_async_copy(ones_v, deg_sh.at[idx_v.at[0]], sems[b]).wait()

    plsc.subcore_barrier()
    pltpu.sync_copy(deg_sh.at[pl.ds(base, RPT)],
                    out_hbm.at[cid, pl.ds(base, RPT)])


@functools.partial(
    pl.kernel,
    out_type=jax.ShapeDtypeStruct((NC, NPAD, 128), jnp.float32),
    mesh=_mesh,
    compiler_params=_sc_params,
    scratch_types=[
        pltpu.VMEM((NCHUNK, CHUNK), jnp.int32),
        pltpu.VMEM((NCHUNK, CHUNK), jnp.int32),
        pltpu.VMEM((ZB, FH), jnp.float32),
        pltpu.VMEM_SHARED((NPAD, FH), jnp.float32),
    ]
    + [pltpu.VMEM((CHUNK, FH), jnp.float32)] * NBUF
    + [pltpu.SemaphoreType.DMA] * (2 * NBUF),
)
def _sc_aggregate(g_hbm, src_hbm, dst_hbm, out_hbm,
                  si_v, di_v, zero_v, acc_sh, *bufs_and_sems):
    rows = bufs_and_sems[:NBUF]
    gsem = bufs_and_sems[NBUF:2 * NBUF]
    ssem = bufs_and_sems[2 * NBUF:]
    cid = lax.axis_index("c")
    sid = lax.axis_index("s")
    base = sid * RPT

    @pl.loop(0, ZB)
    def _(r):
        @pl.loop(0, FH, step=16)
        def _(cc):
            zero_v[r, pl.ds(cc, 16)] = jnp.zeros((16,), jnp.float32)

    pltpu.sync_copy(src_hbm.at[cid, sid], si_v)
    pltpu.sync_copy(dst_hbm.at[sid], di_v)

    @pl.loop(0, RPT // ZB)
    def _(z):
        pltpu.sync_copy(zero_v, acc_sh.at[pl.ds(base + z * ZB, ZB)])

    plsc.subcore_barrier()

    @pl.loop(0, NCHUNK, step=NBUF)
    def _(j):
        for b in range(NBUF):
            @pl.when(j > 0)
            def _():
                pltpu.make_async_copy(rows[b], acc_sh.at[di_v.at[0]],
                                      ssem[b]).wait()
            pltpu.async_copy(g_hbm.at[si_v.at[j + b]], rows[b], gsem[b])
        for b in range(NBUF):
            pltpu.make_async_copy(g_hbm.at[si_v.at[0]], rows[b],
                                  gsem[b]).wait()
            pltpu.async_copy(rows[b], acc_sh.at[di_v.at[j + b]],
                             ssem[b], add=True)

    for b in range(NBUF):
        pltpu.make_async_copy(rows[b], acc_sh.at[di_v.at[0]],
                              ssem[b]).wait()

    plsc.subcore_barrier()
    pltpu.sync_copy(acc_sh.at[pl.ds(base, RPT)],
                    out_hbm.at[cid, pl.ds(base, RPT), pl.ds(0, FH)])


def _tc_matmul(x, W1):
    def body(x_ref, w_ref, h_ref):
        h_ref[...] = jnp.dot(x_ref[...], w_ref[...],
                             preferred_element_type=jnp.float32,
                             precision=lax.Precision.HIGHEST)

    blk = 2000
    return pl.pallas_call(
        body,
        grid=(N // blk,),
        in_specs=[
            pl.BlockSpec((blk, F), lambda i: (i, 0)),
            pl.BlockSpec((F, F), lambda i: (0, 0)),
        ],
        out_specs=pl.BlockSpec((blk, F), lambda i: (i, 0)),
        out_shape=jax.ShapeDtypeStruct((N, F), jnp.float32),
    )(x, W1)


def _tc_scale(h, hist):
    def body(h_ref, hi_ref, g_ref):
        deg = 1.0 + hi_ref[0, :, 0] + hi_ref[1, :, 0]
        dinv = lax.rsqrt(deg)
        g = h_ref[...] * dinv[:, None]
        g_ref[0] = g[:, :FH]
        g_ref[1] = g[:, FH:]

    blk = 2000
    return pl.pallas_call(
        body,
        grid=(N // blk,),
        in_specs=[
            pl.BlockSpec((blk, F), lambda i: (i, 0)),
            pl.BlockSpec((NC, blk, 16), lambda i: (0, i, 0)),
        ],
        out_specs=pl.BlockSpec((2, blk, FH), lambda i: (0, i, 0)),
        out_shape=jax.ShapeDtypeStruct((2, N, FH), jnp.float32),
    )(h, hist)


def _tc_finish(p, g, hist, b2):
    def body(p_ref, g_ref, h_ref, b_ref, o_ref):
        deg = 1.0 + h_ref[0, :, 0] + h_ref[1, :, 0]
        dinv = lax.rsqrt(deg)
        s0 = p_ref[0, :, :FH] + g_ref[0]
        s1 = p_ref[1, :, :FH] + g_ref[1]
        s = jnp.concatenate([s0, s1], axis=1)
        o_ref[...] = jnp.maximum(s * dinv[:, None] + b_ref[...], 0.0)

    blk = 2000
    return pl.pallas_call(
        body,
        grid=(N // blk,),
        in_specs=[
            pl.BlockSpec((NC, blk, 128), lambda i: (0, i, 0)),
            pl.BlockSpec((2, blk, FH), lambda i: (0, i, 0)),
            pl.BlockSpec((NC, blk, 16), lambda i: (0, i, 0)),
            pl.BlockSpec((1, F), lambda i: (0, 0)),
        ],
        out_specs=pl.BlockSpec((blk, F), lambda i: (i, 0)),
        out_shape=jax.ShapeDtypeStruct((N, F), jnp.float32),
    )(p, g, hist, b2)


def kernel(x, edge_index, W1, b1):
    npad = EPAD - E
    ei = edge_index.astype(jnp.int32).reshape(2, E // 128, 128)
    ar = jnp.arange(npad, dtype=jnp.int32).reshape(npad // 128, 128)
    pads = jnp.stack([ar % 128, N + (ar % (NPAD - N))])
    eip = jnp.concatenate([ei, pads], axis=1)
    srcb = jnp.stack([eip[0], eip[0] + N]).reshape(NC, NS, NCHUNK, CHUNK)
    dst3 = eip[1].reshape(NS, NCHUNK, CHUNK)

    hist = _sc_degree(dst3)
    h = _tc_matmul(x, W1)
    g = _tc_scale(h, hist)
    p = _sc_aggregate(g.reshape(2 * N, FH), srcb, dst3)
    return _tc_finish(p, g, hist, b1.reshape(1, F))


# speedup vs baseline: 44.1838x; 1.0243x over previous
import functools

import jax
import jax.numpy as jnp
from jax import lax
from jax.experimental import pallas as pl
from jax.experimental.pallas import tpu as pltpu
from jax.experimental.pallas import tpu_sc as plsc

N = 10000
E = 320000
F = 128
FH = F // 2
NC, NS = 2, 16
CHUNK = 128
NCHUNK = 160
NCHDEG = 80
NBUF = 5
NBUFD = 8
ZB = 128
PER_T = CHUNK * NCHUNK
EPAD = NS * PER_T
NPAD = 10240
RPT = NPAD // NS

_mesh = plsc.VectorSubcoreMesh(core_axis_name="c", subcore_axis_name="s")
_sc_params = pltpu.CompilerParams(use_tc_tiling_on_sc=False)


@functools.partial(
    pl.kernel,
    out_type=jax.ShapeDtypeStruct((NC, NPAD, 128), jnp.float32),
    mesh=_mesh,
    compiler_params=_sc_params,
    scratch_types=[
        pltpu.VMEM((NCHDEG, CHUNK), jnp.int32),
        pltpu.VMEM((CHUNK, 16), jnp.float32),
        pltpu.VMEM((ZB, 16), jnp.float32),
        pltpu.VMEM_SHARED((NPAD, 16), jnp.float32),
    ]
    + [pltpu.SemaphoreType.DMA] * NBUFD,
)
def _sc_degree(dst_hbm, out_hbm, idx_v, ones_v, zero_v, deg_sh, *sems):
    cid = lax.axis_index("c")
    sid = lax.axis_index("s")

    @pl.loop(0, CHUNK)
    def _(r):
        ones_v[r, :] = jnp.ones((16,), jnp.float32)

    @pl.loop(0, ZB)
    def _(r):
        zero_v[r, :] = jnp.zeros((16,), jnp.float32)

    base = sid * RPT
    @pl.loop(0, RPT // ZB)
    def _(z):
        pltpu.sync_copy(zero_v, deg_sh.at[pl.ds(base + z * ZB, ZB)])

    plsc.subcore_barrier()
    pltpu.sync_copy(dst_hbm.at[sid, pl.ds(cid * NCHDEG, NCHDEG)], idx_v)

    @pl.loop(0, NCHDEG, step=NBUFD)
    def _(j):
        for b in range(NBUFD):
            @pl.when(j > 0)
            def _():
                pltpu.make_async_copy(ones_v, deg_sh.at[idx_v.at[0]],
                                      sems[b]).wait()
            pltpu.async_copy(ones_v, deg_sh.at[idx_v.at[j + b]],
                             sems[b], add=True)

    for b in range(NBUFD):
        pltpu.make_async_copy(ones_v, deg_sh.at[idx_v.at[0]], sems[b]).wait()

    plsc.subcore_barrier()
    pltpu.sync_copy(deg_sh.at[pl.ds(base, RPT)],
                    out_hbm.at[cid, pl.ds(base, RPT), pl.ds(0, 16)])


@functools.partial(
    pl.kernel,
    out_type=jax.ShapeDtypeStruct((NC, NPAD, 128), jnp.float32),
    mesh=_mesh,
    compiler_params=_sc_params,
    scratch_types=[
        pltpu.VMEM((NCHUNK, CHUNK), jnp.int32),
        pltpu.VMEM((NCHUNK, CHUNK), jnp.int32),
        pltpu.VMEM((ZB, FH), jnp.float32),
        pltpu.VMEM_SHARED((NPAD, FH), jnp.float32),
    ]
    + [pltpu.VMEM((CHUNK, FH), jnp.float32)] * NBUF
    + [pltpu.SemaphoreType.DMA] * (2 * NBUF),
)
def _sc_aggregate(g_hbm, src_hbm, dst_hbm, out_hbm,
                  si_v, di_v, zero_v, acc_sh, *bufs_and_sems):
    rows = bufs_and_sems[:NBUF]
    gsem = bufs_and_sems[NBUF:2 * NBUF]
    ssem = bufs_and_sems[2 * NBUF:]
    cid = lax.axis_index("c")
    sid = lax.axis_index("s")
    base = sid * RPT

    @pl.loop(0, ZB)
    def _(r):
        @pl.loop(0, FH, step=16)
        def _(cc):
            zero_v[r, pl.ds(cc, 16)] = jnp.zeros((16,), jnp.float32)

    pltpu.sync_copy(src_hbm.at[cid, sid], si_v)
    pltpu.sync_copy(dst_hbm.at[sid], di_v)

    @pl.loop(0, RPT // ZB)
    def _(z):
        pltpu.sync_copy(zero_v, acc_sh.at[pl.ds(base + z * ZB, ZB)])

    plsc.subcore_barrier()

    @pl.loop(0, NCHUNK, step=NBUF)
    def _(j):
        for b in range(NBUF):
            @pl.when(j > 0)
            def _():
                pltpu.make_async_copy(rows[b], acc_sh.at[di_v.at[0]],
                                      ssem[b]).wait()
            pltpu.async_copy(g_hbm.at[si_v.at[j + b]], rows[b], gsem[b])
        for b in range(NBUF):
            pltpu.make_async_copy(g_hbm.at[si_v.at[0]], rows[b],
                                  gsem[b]).wait()
            pltpu.async_copy(rows[b], acc_sh.at[di_v.at[j + b]],
                             ssem[b], add=True)

    for b in range(NBUF):
        pltpu.make_async_copy(rows[b], acc_sh.at[di_v.at[0]],
                              ssem[b]).wait()

    plsc.subcore_barrier()
    pltpu.sync_copy(acc_sh.at[pl.ds(base, RPT)],
                    out_hbm.at[cid, pl.ds(base, RPT), pl.ds(0, FH)])


def _tc_matmul(x, W1):
    def body(x_ref, w_ref, h_ref):
        h_ref[...] = jnp.dot(x_ref[...], w_ref[...],
                             preferred_element_type=jnp.float32,
                             precision=lax.Precision.HIGHEST)

    blk = 2000
    return pl.pallas_call(
        body,
        grid=(N // blk,),
        in_specs=[
            pl.BlockSpec((blk, F), lambda i: (i, 0)),
            pl.BlockSpec((F, F), lambda i: (0, 0)),
        ],
        out_specs=pl.BlockSpec((blk, F), lambda i: (i, 0)),
        out_shape=jax.ShapeDtypeStruct((N, F), jnp.float32),
    )(x, W1)


def _tc_scale(h, hist):
    def body(h_ref, hi_ref, g_ref):
        deg = 1.0 + hi_ref[0, :, 0] + hi_ref[1, :, 0]
        dinv = lax.rsqrt(deg)
        g = h_ref[...] * dinv[:, None]
        g_ref[0] = g[:, :FH]
        g_ref[1] = g[:, FH:]

    blk = 2000
    return pl.pallas_call(
        body,
        grid=(N // blk,),
        in_specs=[
            pl.BlockSpec((blk, F), lambda i: (i, 0)),
            pl.BlockSpec((NC, blk, 128), lambda i: (0, i, 0)),
        ],
        out_specs=pl.BlockSpec((2, blk, FH), lambda i: (0, i, 0)),
        out_shape=jax.ShapeDtypeStruct((2, N, FH), jnp.float32),
    )(h, hist)


def _tc_finish(p, g, hist, b2):
    def body(p_ref, g_ref, h_ref, b_ref, o_ref):
        deg = 1.0 + h_ref[0, :, 0] + h_ref[1, :, 0]
        dinv = lax.rsqrt(deg)
        s0 = p_ref[0, :, :FH] + g_ref[0]
        s1 = p_ref[1, :, :FH] + g_ref[1]
        s = jnp.concatenate([s0, s1], axis=1)
        o_ref[...] = jnp.maximum(s * dinv[:, None] + b_ref[...], 0.0)

    blk = 2000
    return pl.pallas_call(
        body,
        grid=(N // blk,),
        in_specs=[
            pl.BlockSpec((NC, blk, 128), lambda i: (0, i, 0)),
            pl.BlockSpec((2, blk, FH), lambda i: (0, i, 0)),
            pl.BlockSpec((NC, blk, 128), lambda i: (0, i, 0)),
            pl.BlockSpec((1, F), lambda i: (0, 0)),
        ],
        out_specs=pl.BlockSpec((blk, F), lambda i: (i, 0)),
        out_shape=jax.ShapeDtypeStruct((N, F), jnp.float32),
    )(p, g, hist, b2)


def kernel(x, edge_index, W1, b1):
    npad = EPAD - E
    ei = edge_index.astype(jnp.int32).reshape(2, E // 128, 128)
    ar = jnp.arange(npad, dtype=jnp.int32).reshape(npad // 128, 128)
    pads = jnp.stack([ar % 128, N + (ar % (NPAD - N))])
    eip = jnp.concatenate([ei, pads], axis=1)
    srcb = jnp.stack([eip[0], eip[0] + N]).reshape(NC, NS, NCHUNK, CHUNK)
    dst3 = eip[1].reshape(NS, NCHUNK, CHUNK)

    hist = _sc_degree(dst3)
    h = _tc_matmul(x, W1)
    g = _tc_scale(h, hist)
    p = _sc_aggregate(g.reshape(2 * N, FH), srcb, dst3)
    return _tc_finish(p, g, hist, b1.reshape(1, F))
